# baseline (speedup 1.0000x reference)
import functools

import jax
import jax.numpy as jnp
from jax import lax
from jax.experimental import pallas as pl
from jax.experimental.pallas import tpu as pltpu
from jax.experimental.pallas import tpu_sc as plsc

B = 8
H = 16
S = 2048
D = 128
ROWS = B * H * S
NW = 32
GROUPS_PER_W = (B * H) // NW
BPC = B // 2
L = 16
HB = 4


def _sc_body(pos_ref, kv_ref, vv_ref, kc_ref, vc_ref, ka_ref, va_ref,
             pos_v, sidx_v, didx_v, kstage_v, vstage_v, kw_v, vw_v, sem):
    c = lax.axis_index("c")
    s = lax.axis_index("s")
    w = c * 16 + s

    gathers = []
    for j in range(GROUPS_PER_W):
        g = (w * GROUPS_PER_W + j) * S
        gathers.append(pltpu.async_copy(kc_ref.at[pl.ds(g, L)],
                                        kstage_v.at[pl.ds(j * L, L)], sem))
        gathers.append(pltpu.async_copy(vc_ref.at[pl.ds(g, L)],
                                        vstage_v.at[pl.ds(j * L, L)], sem))
    for cp in gathers:
        cp.wait()
    r = w * GROUPS_PER_W * L
    ka = pltpu.async_copy(kstage_v, ka_ref.at[pl.ds(r, GROUPS_PER_W * L)], sem)
    va = pltpu.async_copy(vstage_v, va_ref.at[pl.ds(r, GROUPS_PER_W * L)], sem)
    ka.wait()
    va.wait()
    plsc.subcore_barrier()

    pltpu.sync_copy(pos_ref.at[c], pos_v)
    pv = pos_v[...]
    bl = c * BPC + lax.iota(jnp.int32, L) % BPC
    sidx_v[...] = bl * H + s
    didx_v[...] = (bl * H + pv - 1) * L + s
    kg = pltpu.async_copy(kv_ref.at[sidx_v], kw_v, sem)
    vg = pltpu.async_copy(vv_ref.at[sidx_v], vw_v, sem)
    kg.wait()
    vg.wait()
    ks = pltpu.async_copy(kw_v, ka_ref.at[didx_v], sem)
    vs = pltpu.async_copy(vw_v, va_ref.at[didx_v], sem)
    ks.wait()
    vs.wait()


def _tc_body(kc_ref, vc_ref, ka_ref, va_ref, ko_ref, vo_ref):
    ko_ref[...] = kc_ref[...]
    vo_ref[...] = vc_ref[...]
    ko_ref[0, :, 0:16, :] = ka_ref[0, :, :, :]
    vo_ref[0, :, 0:16, :] = va_ref[0, :, :, :]


def kernel(input_pos, k_val, v_val, k_cache, v_cache):
    posA = jnp.tile(input_pos.reshape(2, BPC), (1, L // BPC))
    kv2 = k_val.reshape(B * H, D)
    vv2 = v_val.reshape(B * H, D)
    kc2 = k_cache.reshape(ROWS, D)
    vc2 = v_cache.reshape(ROWS, D)

    mesh = plsc.VectorSubcoreMesh(core_axis_name="c", subcore_axis_name="s",
                                  num_cores=2)
    sc_run = functools.partial(
        pl.kernel,
        out_type=[
            jax.ShapeDtypeStruct((B * H * L, D), jnp.float32),
            jax.ShapeDtypeStruct((B * H * L, D), jnp.float32),
        ],
        mesh=mesh,
        scratch_types=[
            pltpu.VMEM((L,), jnp.int32),
            pltpu.VMEM((L,), jnp.int32),
            pltpu.VMEM((L,), jnp.int32),
            pltpu.VMEM((GROUPS_PER_W * L, D), jnp.float32),
            pltpu.VMEM((GROUPS_PER_W * L, D), jnp.float32),
            pltpu.VMEM((L, D), jnp.float32),
            pltpu.VMEM((L, D), jnp.float32),
            pltpu.SemaphoreType.DMA,
        ],
    )(_sc_body)
    ka2, va2 = sc_run(posA, kv2, vv2, kc2, vc2)
    ka = ka2.reshape(B, H, L, D)
    va = va2.reshape(B, H, L, D)

    k_out, v_out = pl.pallas_call(
        _tc_body,
        grid=(B, H // HB),
        in_specs=[
            pl.BlockSpec((1, HB, S, D), lambda b, h: (b, h, 0, 0)),
            pl.BlockSpec((1, HB, S, D), lambda b, h: (b, h, 0, 0)),
            pl.BlockSpec((1, HB, L, D), lambda b, h: (b, h, 0, 0)),
            pl.BlockSpec((1, HB, L, D), lambda b, h: (b, h, 0, 0)),
        ],
        out_specs=[
            pl.BlockSpec((1, HB, S, D), lambda b, h: (b, h, 0, 0)),
            pl.BlockSpec((1, HB, S, D), lambda b, h: (b, h, 0, 0)),
        ],
        out_shape=[
            jax.ShapeDtypeStruct((B, H, S, D), jnp.float32),
            jax.ShapeDtypeStruct((B, H, S, D), jnp.float32),
        ],
    )(k_cache, v_cache, ka, va)
    return (k_out, v_out)
